# Initial kernel scaffold; baseline (speedup 1.0000x reference)
import jax
import jax.numpy as jnp
from jax.experimental import pallas as pl


def kernel(x, W, b):
    raise NotImplementedError("write your pallas kernel here")


# speedup vs baseline: 1.1051x; 1.1051x over previous
import functools

import jax
import jax.numpy as jnp
from jax.experimental import pallas as pl
from jax.experimental.pallas import tpu as pltpu

B, D, E = 32768, 4096, 64
TOP_K = 8
BT = 512


def _router_block(x_ref, w_ref, b_ref, topp_ref, topi_ref, allp_ref):
    x_hi = x_ref[...].astype(jnp.bfloat16)

    dims = (((1,), (0,)), ((), ()))
    logits = jax.lax.dot_general(x_hi, w_ref[...], dims,
                                 preferred_element_type=jnp.float32)
    logits += b_ref[...]

    m = jnp.max(logits, axis=1, keepdims=True)
    e = jnp.exp(logits - m)
    s = jnp.sum(e, axis=1, keepdims=True)
    probs = e / s
    allp_ref[...] = probs

    iota = jax.lax.broadcasted_iota(jnp.int32, (BT, E), 1)
    p = probs
    vals = []
    idxs = []
    for _ in range(TOP_K):
        mk = jnp.max(p, axis=1, keepdims=True)
        idx = jnp.min(jnp.where(p == mk, iota, E), axis=1, keepdims=True)
        vals.append(mk)
        idxs.append(idx)
        p = jnp.where(iota == idx, -jnp.inf, p)

    tv = jnp.concatenate(vals, axis=1)
    ti = jnp.concatenate(idxs, axis=1)
    norm = jnp.sum(tv, axis=1, keepdims=True) + 1e-8
    topp_ref[...] = tv / norm
    topi_ref[...] = ti


@jax.jit
def kernel(x, W, b):
    w_hi = W.T.astype(jnp.bfloat16)
    b2 = b.reshape(1, E).astype(jnp.float32)

    grid = (B // BT,)
    out_shape = (
        jax.ShapeDtypeStruct((B, TOP_K), jnp.float32),
        jax.ShapeDtypeStruct((B, TOP_K), jnp.int32),
        jax.ShapeDtypeStruct((B, E), jnp.float32),
    )
    topp, topi, allp = pl.pallas_call(
        _router_block,
        grid=grid,
        in_specs=[
            pl.BlockSpec((BT, D), lambda i: (i, 0)),
            pl.BlockSpec((D, E), lambda i: (0, 0)),
            pl.BlockSpec((1, E), lambda i: (0, 0)),
        ],
        out_specs=(
            pl.BlockSpec((BT, TOP_K), lambda i: (i, 0)),
            pl.BlockSpec((BT, TOP_K), lambda i: (i, 0)),
            pl.BlockSpec((BT, E), lambda i: (i, 0)),
        ),
        out_shape=out_shape,
        compiler_params=pltpu.CompilerParams(
            dimension_semantics=("parallel",),
        ),
    )(x, w_hi, b2)
    return topp, topi, allp


# speedup vs baseline: 1.3566x; 1.2276x over previous
import functools

import jax
import jax.numpy as jnp
from jax.experimental import pallas as pl
from jax.experimental.pallas import tpu as pltpu

B, D, E = 32768, 4096, 64
TOP_K = 8
BT = 512


def _router_block(x_ref, w_ref, b_ref, topp_ref, topi_ref, allp_ref):
    x_hi = x_ref[...].astype(jnp.bfloat16)

    dims = (((1,), (0,)), ((), ()))
    logits = jax.lax.dot_general(x_hi, w_ref[...], dims,
                                 preferred_element_type=jnp.float32)
    logits += b_ref[...]

    m = jnp.max(logits, axis=1, keepdims=True)
    q = logits - m
    e = jnp.exp(q)
    s = jnp.sum(e, axis=1, keepdims=True)
    probs = e / s
    allp_ref[...] = probs

    qt = q.T
    iota = jax.lax.broadcasted_iota(jnp.int32, (E, BT), 0)
    kq = (jnp.maximum(qt, -8.0) * (2.0 ** 22)).astype(jnp.int32)
    key = kq * 64 + (63 - iota)
    int_min = jnp.int32(-(2 ** 31))

    mks = []
    for _ in range(TOP_K):
        mk = jnp.max(key, axis=0, keepdims=True)
        key = jnp.where(key == mk, int_min, key)
        mks.append(mk)

    mkt = jnp.concatenate(mks, axis=0)
    tit = 63 - (mkt & 63)
    qf = (mkt >> 6).astype(jnp.float32) * (2.0 ** -22)
    tvt = jnp.exp(qf)

    tv = tvt.T / s
    norm = jnp.sum(tv, axis=1, keepdims=True) + 1e-8
    topp_ref[...] = tv / norm
    topi_ref[...] = tit.T


@jax.jit
def kernel(x, W, b):
    w_hi = W.T.astype(jnp.bfloat16)
    b2 = b.reshape(1, E).astype(jnp.float32)

    grid = (B // BT,)
    out_shape = (
        jax.ShapeDtypeStruct((B, TOP_K), jnp.float32),
        jax.ShapeDtypeStruct((B, TOP_K), jnp.int32),
        jax.ShapeDtypeStruct((B, E), jnp.float32),
    )
    topp, topi, allp = pl.pallas_call(
        _router_block,
        grid=grid,
        in_specs=[
            pl.BlockSpec((BT, D), lambda i: (i, 0)),
            pl.BlockSpec((D, E), lambda i: (0, 0)),
            pl.BlockSpec((1, E), lambda i: (0, 0)),
        ],
        out_specs=(
            pl.BlockSpec((BT, TOP_K), lambda i: (i, 0)),
            pl.BlockSpec((BT, TOP_K), lambda i: (i, 0)),
            pl.BlockSpec((BT, E), lambda i: (i, 0)),
        ),
        out_shape=out_shape,
        compiler_params=pltpu.CompilerParams(
            dimension_semantics=("parallel",),
        ),
    )(x, w_hi, b2)
    return topp, topi, allp


# speedup vs baseline: 1.4504x; 1.0691x over previous
import functools

import jax
import jax.numpy as jnp
from jax.experimental import pallas as pl
from jax.experimental.pallas import tpu as pltpu

B, D, E = 32768, 4096, 64
TOP_K = 8
BT = 1024


def _router_block(x_ref, w_ref, b_ref, topp_ref, topi_ref, allp_ref):
    x_hi = x_ref[...].astype(jnp.bfloat16)

    dims = (((1,), (0,)), ((), ()))
    logits = jax.lax.dot_general(x_hi, w_ref[...], dims,
                                 preferred_element_type=jnp.float32)
    logits += b_ref[...]

    m = jnp.max(logits, axis=1, keepdims=True)
    q = logits - m
    e = jnp.exp(q)
    s = jnp.sum(e, axis=1, keepdims=True)
    probs = e / s
    allp_ref[...] = probs

    qt = q.T
    iota = jax.lax.broadcasted_iota(jnp.int32, (E, BT), 0)
    kq = (jnp.maximum(qt, -8.0) * (2.0 ** 22)).astype(jnp.int32)
    key = kq * 64 + (63 - iota)
    int_min = jnp.int32(-(2 ** 31))

    mks = []
    for _ in range(TOP_K):
        mk = jnp.max(key, axis=0, keepdims=True)
        key = jnp.where(key == mk, int_min, key)
        mks.append(mk)

    mkt = jnp.concatenate(mks, axis=0)
    tit = 63 - (mkt & 63)
    qf = (mkt >> 6).astype(jnp.float32) * (2.0 ** -22)
    tvt = jnp.exp(qf)

    tv = tvt.T / s
    norm = jnp.sum(tv, axis=1, keepdims=True) + 1e-8
    topp_ref[...] = tv / norm
    topi_ref[...] = tit.T


@jax.jit
def kernel(x, W, b):
    w_hi = W.T.astype(jnp.bfloat16)
    b2 = b.reshape(1, E).astype(jnp.float32)

    grid = (B // BT,)
    out_shape = (
        jax.ShapeDtypeStruct((B, TOP_K), jnp.float32),
        jax.ShapeDtypeStruct((B, TOP_K), jnp.int32),
        jax.ShapeDtypeStruct((B, E), jnp.float32),
    )
    topp, topi, allp = pl.pallas_call(
        _router_block,
        grid=grid,
        in_specs=[
            pl.BlockSpec((BT, D), lambda i: (i, 0)),
            pl.BlockSpec((D, E), lambda i: (0, 0)),
            pl.BlockSpec((1, E), lambda i: (0, 0)),
        ],
        out_specs=(
            pl.BlockSpec((BT, TOP_K), lambda i: (i, 0)),
            pl.BlockSpec((BT, TOP_K), lambda i: (i, 0)),
            pl.BlockSpec((BT, E), lambda i: (i, 0)),
        ),
        out_shape=out_shape,
        compiler_params=pltpu.CompilerParams(
            dimension_semantics=("parallel",),
        ),
    )(x, w_hi, b2)
    return topp, topi, allp


# speedup vs baseline: 1.4609x; 1.0072x over previous
import jax
import jax.numpy as jnp
from jax.experimental import pallas as pl
from jax.experimental.pallas import tpu as pltpu

B, D, E = 32768, 4096, 64
TOP_K = 8
BT = 1024
NC = 4
CH = BT // NC


def _chunk_epilogue(logits, topp_ref, topi_ref, allp_ref, c):
    m = jnp.max(logits, axis=1, keepdims=True)
    q = logits - m
    e = jnp.exp(q)
    s = jnp.sum(e, axis=1, keepdims=True)
    allp_ref[pl.ds(c * CH, CH), :] = e / s

    qt = q.T
    iota = jax.lax.broadcasted_iota(jnp.int32, (E, CH), 0)
    kq = (jnp.maximum(qt, -8.0) * (2.0 ** 22)).astype(jnp.int32)
    key = kq * 64 + (63 - iota)
    int_min = jnp.int32(-(2 ** 31))

    mks = []
    for _ in range(TOP_K):
        mk = jnp.max(key, axis=0, keepdims=True)
        key = jnp.where(key == mk, int_min, key)
        mks.append(mk)

    mkt = jnp.concatenate(mks, axis=0)
    tit = 63 - (mkt & 63)
    qf = (mkt >> 6).astype(jnp.float32) * (2.0 ** -22)
    tvt = jnp.exp(qf)

    tv = tvt.T / s
    norm = jnp.sum(tv, axis=1, keepdims=True) + 1e-8
    topp_ref[pl.ds(c * CH, CH), :] = tv / norm
    topi_ref[pl.ds(c * CH, CH), :] = tit.T


def _router_block(x_ref, w_ref, b_ref, topp_ref, topi_ref, allp_ref):
    dims = (((1,), (0,)), ((), ()))
    w = w_ref[...]
    bias = b_ref[...]
    for c in range(NC):
        x_hi = x_ref[pl.ds(c * CH, CH), :].astype(jnp.bfloat16)
        logits = jax.lax.dot_general(x_hi, w, dims,
                                     preferred_element_type=jnp.float32)
        logits += bias
        _chunk_epilogue(logits, topp_ref, topi_ref, allp_ref, c)


@jax.jit
def kernel(x, W, b):
    w_hi = W.T.astype(jnp.bfloat16)
    b2 = b.reshape(1, E).astype(jnp.float32)

    grid = (B // BT,)
    out_shape = (
        jax.ShapeDtypeStruct((B, TOP_K), jnp.float32),
        jax.ShapeDtypeStruct((B, TOP_K), jnp.int32),
        jax.ShapeDtypeStruct((B, E), jnp.float32),
    )
    topp, topi, allp = pl.pallas_call(
        _router_block,
        grid=grid,
        in_specs=[
            pl.BlockSpec((BT, D), lambda i: (i, 0)),
            pl.BlockSpec((D, E), lambda i: (0, 0)),
            pl.BlockSpec((1, E), lambda i: (0, 0)),
        ],
        out_specs=(
            pl.BlockSpec((BT, TOP_K), lambda i: (i, 0)),
            pl.BlockSpec((BT, TOP_K), lambda i: (i, 0)),
            pl.BlockSpec((BT, E), lambda i: (i, 0)),
        ),
        out_shape=out_shape,
        compiler_params=pltpu.CompilerParams(
            dimension_semantics=("parallel",),
        ),
    )(x, w_hi, b2)
    return topp, topi, allp
